# baseline (speedup 1.0000x reference)
import functools

import jax
import jax.numpy as jnp
from jax import lax
from jax.experimental import pallas as pl
from jax.experimental.pallas import tpu as pltpu
from jax.experimental.pallas import tpu_sc as plsc

B, T, DIM = 1, 2048, 1024
HDIM = 4 * DIM
E = 8
TOPK = 2
N = B * T
TILE = 256
GT = (TOPK * N) // TILE + E
NS = GT * TILE
HBLK = 1024
HB = HDIM // HBLK
AR = 32
TR = AR // TOPK


def _router_kernel(x_ref, rw_ref, counts_ref, psum_ref, zsum_ref,
                   tw_ref, posr_ref, pc_ref, te_ref, tv_ref, rc_ref, xb_ref):
    x = x_ref[...]
    xb_ref[...] = x.astype(jnp.bfloat16)
    logits = jax.lax.dot_general(
        x, rw_ref[...], (((1,), (1,)), ((), ())),
        preferred_element_type=jnp.float32)
    m = jnp.max(logits, axis=-1, keepdims=True)
    ex = jnp.exp(logits - m)
    se = jnp.sum(ex, axis=-1, keepdims=True)
    probs = ex / se

    lane = jax.lax.broadcasted_iota(jnp.int32, (N, E), 1)
    m1 = jnp.max(probs, axis=-1, keepdims=True)
    i1 = jnp.min(jnp.where(probs == m1, lane, E), axis=-1, keepdims=True)
    oh1 = (lane == i1)
    pm = jnp.where(oh1, -jnp.inf, probs)
    m2 = jnp.max(pm, axis=-1, keepdims=True)
    i2 = jnp.min(jnp.where(pm == m2, lane, E), axis=-1, keepdims=True)
    oh2 = (lane == i2)
    denom = m1 + m2
    tw_ref[...] = jnp.concatenate([m1 / denom, m2 / denom], axis=1)

    counts_ref[...] = jnp.sum((oh1 | oh2).astype(jnp.float32), axis=0,
                              keepdims=True)
    psum_ref[...] = jnp.sum(probs, axis=0, keepdims=True)
    lse = jnp.log(se[:, 0]) + m[:, 0]
    zsum_ref[...] = jnp.sum(jnp.square(lse))[None, None]

    e_mat = jnp.concatenate(
        [jnp.reshape(i1[:, 0], (TR, 128)), jnp.reshape(i2[:, 0], (TR, 128))],
        axis=0)

    ci = jax.lax.broadcasted_iota(jnp.int32, (128, 128), 0)
    cj = jax.lax.broadcasted_iota(jnp.int32, (128, 128), 1)
    U = (ci < cj).astype(jnp.float32)
    ri = jax.lax.broadcasted_iota(jnp.int32, (AR, AR), 0)
    rj = jax.lax.broadcasted_iota(jnp.int32, (AR, AR), 1)
    VL = (rj < ri).astype(jnp.float32)

    rank = jnp.zeros((AR, 128), jnp.float32)
    counts_list = []
    for e in range(E):
        mk = (e_mat == e).astype(jnp.float32)
        pfx = jax.lax.dot_general(mk, U, (((1,), (0,)), ((), ())),
                                  preferred_element_type=jnp.float32)
        rowsum = jnp.sum(mk, axis=1, keepdims=True)
        rowpre = jax.lax.dot_general(VL, rowsum, (((1,), (0,)), ((), ())),
                                     preferred_element_type=jnp.float32)
        rank = rank + mk * (rowpre + pfx)
        counts_list.append(jnp.sum(rowsum)[None, None])
    cvec = jnp.concatenate(counts_list, axis=1)
    padded = jnp.ceil(cvec / TILE) * TILE
    ei = jax.lax.broadcasted_iota(jnp.int32, (E, E), 0)
    ej = jax.lax.broadcasted_iota(jnp.int32, (E, E), 1)
    offp = jnp.sum(jnp.where(ej < ei, padded, 0.0), axis=1,
                   keepdims=True)
    total_pad = jnp.sum(padded)

    off_a = jnp.zeros((AR, 128), jnp.float32)
    for e in range(E):
        off_a = off_a + jnp.where(e_mat == e, offp[e, 0], 0.0)
    p_f = off_a + rank

    ii = jax.lax.broadcasted_iota(jnp.int32, (N, TR), 0)
    rr = jax.lax.broadcasted_iota(jnp.int32, (N, TR), 1)
    Asel = ((ii >= rr * 128) & (ii < rr * 128 + 128)).astype(jnp.float32)
    rcol = jax.lax.broadcasted_iota(jnp.int32, (TR, 1), 0).astype(jnp.float32)
    idiv = jax.lax.dot_general(Asel, rcol, (((1,), (0,)), ((), ())),
                               preferred_element_type=jnp.float32)
    icol = jax.lax.broadcasted_iota(jnp.int32, (N, 1), 0).astype(jnp.float32)
    imod = icol - 128.0 * idiv
    ccr = jax.lax.broadcasted_iota(jnp.int32, (1, 128), 1).astype(jnp.float32)
    B2 = (imod == ccr).astype(jnp.float32)

    irow = jax.lax.broadcasted_iota(jnp.int32, (1, N), 1).astype(jnp.float32)
    idiv_r = jax.lax.dot_general(rcol, Asel, (((0,), (1,)), ((), ())),
                                 preferred_element_type=jnp.float32)
    imod_r = irow - 128.0 * idiv_r
    ccc = jax.lax.broadcasted_iota(jnp.int32, (128, 1), 0).astype(jnp.float32)
    B2t = (imod_r == ccc).astype(jnp.float32)

    pcs, prs = [], []
    for k in range(TOPK):
        Mk = p_f[k * TR:(k + 1) * TR]
        AM = jax.lax.dot_general(Asel, Mk, (((1,), (0,)), ((), ())),
                                 preferred_element_type=jnp.float32,
                                 precision=jax.lax.Precision.HIGHEST)
        pcs.append(jnp.sum(AM * B2, axis=1, keepdims=True))
        MA = jax.lax.dot_general(Mk, Asel, (((0,), (1,)), ((), ())),
                                 preferred_element_type=jnp.float32,
                                 precision=jax.lax.Precision.HIGHEST)
        prs.append(jnp.sum(MA * B2t, axis=0, keepdims=True))
    pc_ref[...] = jnp.concatenate(pcs, axis=1).astype(jnp.int32)
    posr_ref[...] = jnp.concatenate(prs, axis=0).astype(jnp.int32)

    tl = jax.lax.broadcasted_iota(
        jnp.int32, (1, 128), 1).astype(jnp.float32) * TILE
    tlc = jnp.minimum(tl, total_pad - 1.0)
    te = jnp.zeros((1, 128), jnp.float32)
    for e in range(1, E):
        te = te + (tlc >= offp[e, 0]).astype(jnp.float32)
    te_ref[...] = te.astype(jnp.int32)
    tv_ref[...] = (tl < total_pad).astype(jnp.int32)
    ce = jnp.zeros((1, 128), jnp.float32)
    for e in range(E):
        ce = ce + jnp.where(te == e, offp[e, 0] + cvec[0, e], 0.0)
    rc_ref[...] = jnp.clip(ce - tl, 0.0, TILE).astype(jnp.int32)


def _group_kernel(te_ref, tv_ref, rc_ref, xs_ref, up_ref, dn_ref,
                  pc_ref, tw_ref, sh_ref, out_ref, acc_ref):
    hb = pl.program_id(0)
    tau = pl.program_id(1)

    @pl.when(tau < GT)
    def _():
        rows = pl.ds(tau * TILE, TILE)
        valid = tv_ref[tau] > 0

        @pl.when(valid)
        def _():
            xt = xs_ref[...].astype(jnp.bfloat16)
            up = up_ref[0].astype(jnp.bfloat16)
            dn = dn_ref[0].astype(jnp.bfloat16)
            h = jnp.dot(xt, up, preferred_element_type=jnp.float32)
            h = jnp.square(jnp.maximum(h, 0.0))
            y = jax.lax.dot_general(
                h.astype(jnp.bfloat16), dn, (((1,), (1,)), ((), ())),
                preferred_element_type=jnp.float32).astype(jnp.bfloat16)

            @pl.when(hb == 0)
            def _():
                acc_ref[rows, :] = y

            @pl.when(jnp.logical_and(hb > 0, hb < HB - 1))
            def _():
                acc_ref[rows, :] += y

            @pl.when(hb == HB - 1)
            def _():
                ri = jax.lax.broadcasted_iota(jnp.int32, (TILE, 1), 0)
                m = ri < rc_ref[tau]
                acc_ref[rows, :] = jnp.where(
                    m, acc_ref[rows, :] + y, jnp.bfloat16(0))

        @pl.when(jnp.logical_and(jnp.logical_not(valid), hb == HB - 1))
        def _():
            acc_ref[rows, :] = jnp.zeros((TILE, DIM), jnp.bfloat16)

    @pl.when(jnp.logical_and(tau >= GT, hb == HB - 1))
    def _():
        trows = pl.ds((tau - GT) * TILE, TILE)
        p1 = pc_ref[trows, 0:1]
        p2 = pc_ref[trows, 1:2]
        w1 = tw_ref[trows, 0:1]
        w2 = tw_ref[trows, 1:2]
        slot = jax.lax.broadcasted_iota(jnp.int32, (TILE, NS), 1)
        C = (jnp.where(slot == p1, w1, 0.0)
             + jnp.where(slot == p2, w2, 0.0)).astype(jnp.bfloat16)
        out_ref[...] = sh_ref[...] + jnp.dot(
            C, acc_ref[...], preferred_element_type=jnp.float32)


_SCI = plsc.get_sparse_core_info()
_NW = _SCI.num_cores * _SCI.num_subcores
_ACH = (TOPK * N) // _NW
_SCH = 64


def _sc_scatter_kernel(x_hbm, pos_hbm, xs_hbm, idx_v, rows_v, sem):
    wid = lax.axis_index("s") * _SCI.num_cores + lax.axis_index("c")
    abase = wid * _ACH
    tbase = abase % N
    for j in range(_ACH // _SCH):
        pltpu.sync_copy(pos_hbm.at[pl.ds(abase + j * _SCH, _SCH)], idx_v)
        pltpu.sync_copy(x_hbm.at[pl.ds(tbase + j * _SCH, _SCH)], rows_v)
        pltpu.async_copy(rows_v, xs_hbm.at[idx_v], sem).wait()


def _shared_kernel(x_ref, up_ref, dn_ref, out_ref, acc_ref):
    hb = pl.program_id(0)
    t = pl.program_id(1)
    rows = pl.ds(t * TILE, TILE)

    xt = x_ref[rows, :]
    up = up_ref[...].astype(jnp.bfloat16)
    dn = dn_ref[...].astype(jnp.bfloat16)
    h = jnp.dot(xt, up, preferred_element_type=jnp.float32)
    h = jnp.square(jnp.maximum(h, 0.0))
    y = jax.lax.dot_general(
        h.astype(jnp.bfloat16), dn, (((1,), (1,)), ((), ())),
        preferred_element_type=jnp.float32)

    @pl.when(hb == 0)
    def _():
        acc_ref[rows, :] = y

    @pl.when(hb > 0)
    def _():
        acc_ref[rows, :] += y

    @pl.when(hb == HB - 1)
    def _():
        out_ref[...] = acc_ref[rows, :]


def kernel(x, router_w, W_shared_up, W_shared_down, W_up, W_down):
    xf = x.reshape(N, DIM)

    counts, p_sum, z_sum, tw, posr, pc, te, tv, rc, x_bf = pl.pallas_call(
        _router_kernel,
        out_shape=[
            jax.ShapeDtypeStruct((1, E), jnp.float32),
            jax.ShapeDtypeStruct((1, E), jnp.float32),
            jax.ShapeDtypeStruct((1, 1), jnp.float32),
            jax.ShapeDtypeStruct((N, TOPK), jnp.float32),
            jax.ShapeDtypeStruct((TOPK, N), jnp.int32),
            jax.ShapeDtypeStruct((N, TOPK), jnp.int32),
            jax.ShapeDtypeStruct((1, 128), jnp.int32),
            jax.ShapeDtypeStruct((1, 128), jnp.int32),
            jax.ShapeDtypeStruct((1, 128), jnp.int32),
            jax.ShapeDtypeStruct((N, DIM), jnp.bfloat16),
        ],
    )(xf, router_w)

    NT2 = N // TILE
    te_flat = te.reshape(128)[:GT + NT2]
    tv_flat = tv.reshape(128)[:GT + NT2]
    rc_flat = rc.reshape(128)[:GT + NT2]

    xs_f = pl.kernel(
        _sc_scatter_kernel,
        out_type=jax.ShapeDtypeStruct((NS, DIM), jnp.float32),
        mesh=plsc.VectorSubcoreMesh(core_axis_name="c", subcore_axis_name="s"),
        scratch_types=[
            pltpu.VMEM((_SCH,), jnp.int32),
            pltpu.VMEM((_SCH, DIM), jnp.float32),
            pltpu.SemaphoreType.DMA,
        ],
    )(xf, posr.reshape(TOPK * N))

    shared_y = pl.pallas_call(
        _shared_kernel,
        grid=(HB, N // TILE),
        in_specs=[
            pl.BlockSpec((N, DIM), lambda hb, t: (0, 0)),
            pl.BlockSpec((DIM, HBLK), lambda hb, t: (0, hb)),
            pl.BlockSpec((DIM, HBLK), lambda hb, t: (0, hb)),
        ],
        out_specs=pl.BlockSpec((TILE, DIM), lambda hb, t: (t, 0)),
        out_shape=jax.ShapeDtypeStruct((N, DIM), jnp.float32),
        scratch_shapes=[pltpu.VMEM((N, DIM), jnp.float32)],
    )(x_bf, W_shared_up, W_shared_down)

    def _tok_map(hb, t, te, tv, rc):
        tt = jnp.where(hb == HB - 1,
                       jnp.clip(t - GT, 0, NT2 - 1), 0)
        return (tt, 0)

    out = pl.pallas_call(
        _group_kernel,
        grid_spec=pltpu.PrefetchScalarGridSpec(
            num_scalar_prefetch=3,
            grid=(HB, GT + NT2),
            in_specs=[
                pl.BlockSpec((TILE, DIM),
                             lambda hb, t, te, tv, rc: (
                                 jnp.minimum(t, GT - 1), 0)),
                pl.BlockSpec((1, DIM, HBLK),
                             lambda hb, t, te, tv, rc: (te[t], 0, hb)),
                pl.BlockSpec((1, DIM, HBLK),
                             lambda hb, t, te, tv, rc: (te[t], 0, hb)),
                pl.BlockSpec((N, TOPK), lambda hb, t, te, tv, rc: (0, 0)),
                pl.BlockSpec((N, TOPK), lambda hb, t, te, tv, rc: (0, 0)),
                pl.BlockSpec((TILE, DIM), _tok_map),
            ],
            out_specs=pl.BlockSpec((TILE, DIM), _tok_map),
            scratch_shapes=[pltpu.VMEM((NS, DIM), jnp.bfloat16)],
        ),
        out_shape=jax.ShapeDtypeStruct((N, DIM), jnp.float32),
    )(te_flat, tv_flat, rc_flat, xs_f, W_up, W_down, pc, tw, shared_y)

    f = counts[0] / (N * TOPK)
    p_mean = p_sum[0] / N
    lb = E * jnp.sum(f * p_mean)
    z = z_sum[0, 0] / N
    return out.reshape(B, T, DIM), lb, z


# speedup vs baseline: 1.0896x; 1.0896x over previous
import jax
import jax.numpy as jnp
from jax.experimental import pallas as pl
from jax.experimental.pallas import tpu as pltpu

B, T, DIM = 1, 2048, 1024
HDIM = 4 * DIM
E = 8
TOPK = 2
N = B * T
TILE = 256
GT = (TOPK * N) // TILE + E
NS = GT * TILE
HBLK = 1024
HB = HDIM // HBLK
AR = 32
TR = AR // TOPK


def _router_kernel(x_ref, rw_ref, counts_ref, psum_ref, zsum_ref,
                   tw_ref, posr_ref, pc_ref, te_ref, tv_ref, xb_ref):
    x = x_ref[...]
    xb_ref[...] = x.astype(jnp.bfloat16)
    logits = jax.lax.dot_general(
        x, rw_ref[...], (((1,), (1,)), ((), ())),
        preferred_element_type=jnp.float32)
    m = jnp.max(logits, axis=-1, keepdims=True)
    ex = jnp.exp(logits - m)
    se = jnp.sum(ex, axis=-1, keepdims=True)
    probs = ex / se

    lane = jax.lax.broadcasted_iota(jnp.int32, (N, E), 1)
    m1 = jnp.max(probs, axis=-1, keepdims=True)
    i1 = jnp.min(jnp.where(probs == m1, lane, E), axis=-1, keepdims=True)
    oh1 = (lane == i1)
    pm = jnp.where(oh1, -jnp.inf, probs)
    m2 = jnp.max(pm, axis=-1, keepdims=True)
    i2 = jnp.min(jnp.where(pm == m2, lane, E), axis=-1, keepdims=True)
    oh2 = (lane == i2)
    denom = m1 + m2
    tw_ref[...] = jnp.concatenate([m1 / denom, m2 / denom], axis=1)

    counts_ref[...] = jnp.sum((oh1 | oh2).astype(jnp.float32), axis=0,
                              keepdims=True)
    psum_ref[...] = jnp.sum(probs, axis=0, keepdims=True)
    lse = jnp.log(se[:, 0]) + m[:, 0]
    zsum_ref[...] = jnp.sum(jnp.square(lse))[None, None]

    e_mat = jnp.concatenate(
        [jnp.reshape(i1[:, 0], (TR, 128)), jnp.reshape(i2[:, 0], (TR, 128))],
        axis=0)

    ci = jax.lax.broadcasted_iota(jnp.int32, (128, 128), 0)
    cj = jax.lax.broadcasted_iota(jnp.int32, (128, 128), 1)
    U = (ci < cj).astype(jnp.float32)
    ri = jax.lax.broadcasted_iota(jnp.int32, (AR, AR), 0)
    rj = jax.lax.broadcasted_iota(jnp.int32, (AR, AR), 1)
    VL = (rj < ri).astype(jnp.float32)

    rank = jnp.zeros((AR, 128), jnp.float32)
    counts_list = []
    for e in range(E):
        mk = (e_mat == e).astype(jnp.float32)
        pfx = jax.lax.dot_general(mk, U, (((1,), (0,)), ((), ())),
                                  preferred_element_type=jnp.float32)
        rowsum = jnp.sum(mk, axis=1, keepdims=True)
        rowpre = jax.lax.dot_general(VL, rowsum, (((1,), (0,)), ((), ())),
                                     preferred_element_type=jnp.float32)
        rank = rank + mk * (rowpre + pfx)
        counts_list.append(jnp.sum(rowsum)[None, None])
    cvec = jnp.concatenate(counts_list, axis=1)
    padded = jnp.ceil(cvec / TILE) * TILE
    ei = jax.lax.broadcasted_iota(jnp.int32, (E, E), 0)
    ej = jax.lax.broadcasted_iota(jnp.int32, (E, E), 1)
    offp = jnp.sum(jnp.where(ej < ei, padded, 0.0), axis=1,
                   keepdims=True)
    total_pad = jnp.sum(padded)

    off_a = jnp.zeros((AR, 128), jnp.float32)
    for e in range(E):
        off_a = off_a + jnp.where(e_mat == e, offp[e, 0], 0.0)
    p_f = off_a + rank

    ii = jax.lax.broadcasted_iota(jnp.int32, (N, TR), 0)
    rr = jax.lax.broadcasted_iota(jnp.int32, (N, TR), 1)
    Asel = ((ii >= rr * 128) & (ii < rr * 128 + 128)).astype(jnp.float32)
    rcol = jax.lax.broadcasted_iota(jnp.int32, (TR, 1), 0).astype(jnp.float32)
    idiv = jax.lax.dot_general(Asel, rcol, (((1,), (0,)), ((), ())),
                               preferred_element_type=jnp.float32)
    icol = jax.lax.broadcasted_iota(jnp.int32, (N, 1), 0).astype(jnp.float32)
    imod = icol - 128.0 * idiv
    ccr = jax.lax.broadcasted_iota(jnp.int32, (1, 128), 1).astype(jnp.float32)
    B2 = (imod == ccr).astype(jnp.float32)

    irow = jax.lax.broadcasted_iota(jnp.int32, (1, N), 1).astype(jnp.float32)
    idiv_r = jax.lax.dot_general(rcol, Asel, (((0,), (1,)), ((), ())),
                                 preferred_element_type=jnp.float32)
    imod_r = irow - 128.0 * idiv_r
    ccc = jax.lax.broadcasted_iota(jnp.int32, (128, 1), 0).astype(jnp.float32)
    B2t = (imod_r == ccc).astype(jnp.float32)

    pcs, prs = [], []
    for k in range(TOPK):
        Mk = p_f[k * TR:(k + 1) * TR]
        AM = jax.lax.dot_general(Asel, Mk, (((1,), (0,)), ((), ())),
                                 preferred_element_type=jnp.float32,
                                 precision=jax.lax.Precision.HIGHEST)
        pcs.append(jnp.sum(AM * B2, axis=1, keepdims=True))
        MA = jax.lax.dot_general(Mk, Asel, (((0,), (1,)), ((), ())),
                                 preferred_element_type=jnp.float32,
                                 precision=jax.lax.Precision.HIGHEST)
        prs.append(jnp.sum(MA * B2t, axis=0, keepdims=True))
    pc_ref[...] = jnp.concatenate(pcs, axis=1).astype(jnp.int32)
    posr_ref[...] = jnp.concatenate(prs, axis=0).astype(jnp.int32)

    tl = jax.lax.broadcasted_iota(
        jnp.int32, (1, 128), 1).astype(jnp.float32) * TILE
    tlc = jnp.minimum(tl, total_pad - 1.0)
    te = jnp.zeros((1, 128), jnp.float32)
    for e in range(1, E):
        te = te + (tlc >= offp[e, 0]).astype(jnp.float32)
    te_ref[...] = te.astype(jnp.int32)
    tv_ref[...] = (tl < total_pad).astype(jnp.int32)


def _group_kernel(te_ref, tv_ref, posr_ref, x_ref, up_ref, dn_ref,
                  pc_ref, tw_ref, sh_ref, out_ref, xs_ref, acc_ref):
    hb = pl.program_id(0)
    tau = pl.program_id(1)

    @pl.when(tau < GT)
    def _():
        rows = pl.ds(tau * TILE, TILE)
        valid = tv_ref[tau] > 0

        @pl.when(jnp.logical_and(valid, hb == 0))
        def _():
            slot = tau * TILE + jax.lax.broadcasted_iota(
                jnp.int32, (TILE, N), 0)
            p1 = posr_ref[0:1, :]
            p2 = posr_ref[1:2, :]
            P = jnp.logical_or(slot == p1, slot == p2).astype(jnp.bfloat16)
            xs_ref[rows, :] = jnp.dot(P, x_ref[...],
                                      preferred_element_type=jnp.float32
                                      ).astype(jnp.bfloat16)

        @pl.when(valid)
        def _():
            xt = xs_ref[rows, :]
            up = up_ref[0].astype(jnp.bfloat16)
            dn = dn_ref[0].astype(jnp.bfloat16)
            h = jnp.dot(xt, up, preferred_element_type=jnp.float32)
            h = jnp.square(jnp.maximum(h, 0.0))
            y = jax.lax.dot_general(
                h.astype(jnp.bfloat16), dn, (((1,), (1,)), ((), ())),
                preferred_element_type=jnp.float32).astype(jnp.bfloat16)

            @pl.when(hb == 0)
            def _():
                acc_ref[rows, :] = y

            @pl.when(hb > 0)
            def _():
                acc_ref[rows, :] += y

        @pl.when(jnp.logical_and(jnp.logical_not(valid), hb == HB - 1))
        def _():
            acc_ref[rows, :] = jnp.zeros((TILE, DIM), jnp.bfloat16)

    @pl.when(jnp.logical_and(tau >= GT, hb == HB - 1))
    def _():
        trows = pl.ds((tau - GT) * TILE, TILE)
        p1 = pc_ref[trows, 0:1]
        p2 = pc_ref[trows, 1:2]
        w1 = tw_ref[trows, 0:1]
        w2 = tw_ref[trows, 1:2]
        slot = jax.lax.broadcasted_iota(jnp.int32, (TILE, NS), 1)
        C = (jnp.where(slot == p1, w1, 0.0)
             + jnp.where(slot == p2, w2, 0.0)).astype(jnp.bfloat16)
        out_ref[...] = sh_ref[...] + jnp.dot(
            C, acc_ref[...], preferred_element_type=jnp.float32)


def _shared_kernel(x_ref, up_ref, dn_ref, out_ref, acc_ref):
    hb = pl.program_id(0)
    t = pl.program_id(1)
    rows = pl.ds(t * TILE, TILE)

    xt = x_ref[rows, :]
    up = up_ref[...].astype(jnp.bfloat16)
    dn = dn_ref[...].astype(jnp.bfloat16)
    h = jnp.dot(xt, up, preferred_element_type=jnp.float32)
    h = jnp.square(jnp.maximum(h, 0.0))
    y = jax.lax.dot_general(
        h.astype(jnp.bfloat16), dn, (((1,), (1,)), ((), ())),
        preferred_element_type=jnp.float32)

    @pl.when(hb == 0)
    def _():
        acc_ref[rows, :] = y

    @pl.when(hb > 0)
    def _():
        acc_ref[rows, :] += y

    @pl.when(hb == HB - 1)
    def _():
        out_ref[...] = acc_ref[rows, :]


def kernel(x, router_w, W_shared_up, W_shared_down, W_up, W_down):
    xf = x.reshape(N, DIM)

    counts, p_sum, z_sum, tw, posr, pc, te, tv, x_bf = pl.pallas_call(
        _router_kernel,
        out_shape=[
            jax.ShapeDtypeStruct((1, E), jnp.float32),
            jax.ShapeDtypeStruct((1, E), jnp.float32),
            jax.ShapeDtypeStruct((1, 1), jnp.float32),
            jax.ShapeDtypeStruct((N, TOPK), jnp.float32),
            jax.ShapeDtypeStruct((TOPK, N), jnp.int32),
            jax.ShapeDtypeStruct((N, TOPK), jnp.int32),
            jax.ShapeDtypeStruct((1, 128), jnp.int32),
            jax.ShapeDtypeStruct((1, 128), jnp.int32),
            jax.ShapeDtypeStruct((N, DIM), jnp.bfloat16),
        ],
    )(xf, router_w)

    NT2 = N // TILE
    te_flat = te.reshape(128)[:GT + NT2]
    tv_flat = tv.reshape(128)[:GT + NT2]

    shared_y = pl.pallas_call(
        _shared_kernel,
        grid=(HB, N // TILE),
        in_specs=[
            pl.BlockSpec((N, DIM), lambda hb, t: (0, 0)),
            pl.BlockSpec((DIM, HBLK), lambda hb, t: (0, hb)),
            pl.BlockSpec((DIM, HBLK), lambda hb, t: (0, hb)),
        ],
        out_specs=pl.BlockSpec((TILE, DIM), lambda hb, t: (t, 0)),
        out_shape=jax.ShapeDtypeStruct((N, DIM), jnp.float32),
        scratch_shapes=[pltpu.VMEM((N, DIM), jnp.float32)],
    )(x_bf, W_shared_up, W_shared_down)

    def _tok_map(hb, t, te, tv):
        tt = jnp.where(hb == HB - 1,
                       jnp.clip(t - GT, 0, NT2 - 1), 0)
        return (tt, 0)

    out = pl.pallas_call(
        _group_kernel,
        grid_spec=pltpu.PrefetchScalarGridSpec(
            num_scalar_prefetch=2,
            grid=(HB, GT + NT2),
            in_specs=[
                pl.BlockSpec((TOPK, N), lambda hb, t, te, tv: (0, 0)),
                pl.BlockSpec((N, DIM), lambda hb, t, te, tv: (0, 0)),
                pl.BlockSpec((1, DIM, HBLK),
                             lambda hb, t, te, tv: (te[t], 0, hb)),
                pl.BlockSpec((1, DIM, HBLK),
                             lambda hb, t, te, tv: (te[t], 0, hb)),
                pl.BlockSpec((N, TOPK), lambda hb, t, te, tv: (0, 0)),
                pl.BlockSpec((N, TOPK), lambda hb, t, te, tv: (0, 0)),
                pl.BlockSpec((TILE, DIM), _tok_map),
            ],
            out_specs=pl.BlockSpec((TILE, DIM), _tok_map),
            scratch_shapes=[pltpu.VMEM((NS, DIM), jnp.bfloat16),
                            pltpu.VMEM((NS, DIM), jnp.bfloat16)],
        ),
        out_shape=jax.ShapeDtypeStruct((N, DIM), jnp.float32),
    )(te_flat, tv_flat, posr, x_bf, W_up, W_down, pc, tw, shared_y)

    f = counts[0] / (N * TOPK)
    p_mean = p_sum[0] / N
    lb = E * jnp.sum(f * p_mean)
    z = z_sum[0, 0] / N
    return out.reshape(B, T, DIM), lb, z


# speedup vs baseline: 1.0962x; 1.0060x over previous
import jax
import jax.numpy as jnp
from jax.experimental import pallas as pl
from jax.experimental.pallas import tpu as pltpu

B, T, DIM = 1, 2048, 1024
HDIM = 4 * DIM
E = 8
TOPK = 2
N = B * T
TILE = 256
GT = (TOPK * N) // TILE + E
NS = GT * TILE
HBLK = 1024
HB = HDIM // HBLK
AR = 32
TR = AR // TOPK


def _router_kernel(x_ref, rw_ref, counts_ref, psum_ref, zsum_ref,
                   tw_ref, posr_ref, pc_ref, te_ref, tv_ref, xb_ref):
    x = x_ref[...]
    xb_ref[...] = x.astype(jnp.bfloat16)
    logits = jax.lax.dot_general(
        x, rw_ref[...], (((1,), (1,)), ((), ())),
        preferred_element_type=jnp.float32)
    m = jnp.max(logits, axis=-1, keepdims=True)
    ex = jnp.exp(logits - m)
    se = jnp.sum(ex, axis=-1, keepdims=True)
    probs = ex / se

    lane = jax.lax.broadcasted_iota(jnp.int32, (N, E), 1)
    m1 = jnp.max(probs, axis=-1, keepdims=True)
    i1 = jnp.min(jnp.where(probs == m1, lane, E), axis=-1, keepdims=True)
    oh1 = (lane == i1)
    pm = jnp.where(oh1, -jnp.inf, probs)
    m2 = jnp.max(pm, axis=-1, keepdims=True)
    i2 = jnp.min(jnp.where(pm == m2, lane, E), axis=-1, keepdims=True)
    oh2 = (lane == i2)
    denom = m1 + m2
    tw_ref[...] = jnp.concatenate([m1 / denom, m2 / denom], axis=1)

    counts_ref[...] = jnp.sum((oh1 | oh2).astype(jnp.float32), axis=0,
                              keepdims=True)
    psum_ref[...] = jnp.sum(probs, axis=0, keepdims=True)
    lse = jnp.log(se[:, 0]) + m[:, 0]
    zsum_ref[...] = jnp.sum(jnp.square(lse))[None, None]

    e_mat = jnp.concatenate(
        [jnp.reshape(i1[:, 0], (TR, 128)), jnp.reshape(i2[:, 0], (TR, 128))],
        axis=0)

    ci = jax.lax.broadcasted_iota(jnp.int32, (128, 128), 0)
    cj = jax.lax.broadcasted_iota(jnp.int32, (128, 128), 1)
    U = (ci < cj).astype(jnp.float32)
    ri = jax.lax.broadcasted_iota(jnp.int32, (AR, AR), 0)
    rj = jax.lax.broadcasted_iota(jnp.int32, (AR, AR), 1)
    VL = (rj < ri).astype(jnp.float32)

    rank = jnp.zeros((AR, 128), jnp.float32)
    counts_list = []
    for e in range(E):
        mk = (e_mat == e).astype(jnp.float32)
        pfx = jax.lax.dot_general(mk, U, (((1,), (0,)), ((), ())),
                                  preferred_element_type=jnp.float32)
        rowsum = jnp.sum(mk, axis=1, keepdims=True)
        rowpre = jax.lax.dot_general(VL, rowsum, (((1,), (0,)), ((), ())),
                                     preferred_element_type=jnp.float32)
        rank = rank + mk * (rowpre + pfx)
        counts_list.append(jnp.sum(rowsum)[None, None])
    cvec = jnp.concatenate(counts_list, axis=1)
    padded = jnp.ceil(cvec / TILE) * TILE
    ei = jax.lax.broadcasted_iota(jnp.int32, (E, E), 0)
    ej = jax.lax.broadcasted_iota(jnp.int32, (E, E), 1)
    offp = jnp.sum(jnp.where(ej < ei, padded, 0.0), axis=1,
                   keepdims=True)
    total_pad = jnp.sum(padded)

    off_a = jnp.zeros((AR, 128), jnp.float32)
    for e in range(E):
        off_a = off_a + jnp.where(e_mat == e, offp[e, 0], 0.0)
    p_f = off_a + rank

    ii = jax.lax.broadcasted_iota(jnp.int32, (N, TR), 0)
    rr = jax.lax.broadcasted_iota(jnp.int32, (N, TR), 1)
    Asel = ((ii >= rr * 128) & (ii < rr * 128 + 128)).astype(jnp.float32)
    rcol = jax.lax.broadcasted_iota(jnp.int32, (TR, 1), 0).astype(jnp.float32)
    idiv = jax.lax.dot_general(Asel, rcol, (((1,), (0,)), ((), ())),
                               preferred_element_type=jnp.float32)
    icol = jax.lax.broadcasted_iota(jnp.int32, (N, 1), 0).astype(jnp.float32)
    imod = icol - 128.0 * idiv
    ccr = jax.lax.broadcasted_iota(jnp.int32, (1, 128), 1).astype(jnp.float32)
    B2 = (imod == ccr).astype(jnp.float32)

    irow = jax.lax.broadcasted_iota(jnp.int32, (1, N), 1).astype(jnp.float32)
    idiv_r = jax.lax.dot_general(rcol, Asel, (((0,), (1,)), ((), ())),
                                 preferred_element_type=jnp.float32)
    imod_r = irow - 128.0 * idiv_r
    ccc = jax.lax.broadcasted_iota(jnp.int32, (128, 1), 0).astype(jnp.float32)
    B2t = (imod_r == ccc).astype(jnp.float32)

    pcs, prs = [], []
    for k in range(TOPK):
        Mk = p_f[k * TR:(k + 1) * TR]
        AM = jax.lax.dot_general(Asel, Mk, (((1,), (0,)), ((), ())),
                                 preferred_element_type=jnp.float32,
                                 precision=jax.lax.Precision.HIGHEST)
        pcs.append(jnp.sum(AM * B2, axis=1, keepdims=True))
        MA = jax.lax.dot_general(Mk, Asel, (((0,), (1,)), ((), ())),
                                 preferred_element_type=jnp.float32,
                                 precision=jax.lax.Precision.HIGHEST)
        prs.append(jnp.sum(MA * B2t, axis=0, keepdims=True))
    pc_ref[...] = jnp.concatenate(pcs, axis=1).astype(jnp.int32)
    posr_ref[...] = jnp.concatenate(prs, axis=0).astype(jnp.int32)

    tl = jax.lax.broadcasted_iota(
        jnp.int32, (1, 128), 1).astype(jnp.float32) * TILE
    tlc = jnp.minimum(tl, total_pad - 1.0)
    te = jnp.zeros((1, 128), jnp.float32)
    for e in range(1, E):
        te = te + (tlc >= offp[e, 0]).astype(jnp.float32)
    te_ref[...] = te.astype(jnp.int32)
    tv_ref[...] = (tl < total_pad).astype(jnp.int32)


def _group_kernel(te_ref, tv_ref, posr_ref, x_ref, up_ref, dn_ref,
                  pc_ref, tw_ref, sh_ref, out_ref, xs_ref, acc_ref):
    hb = pl.program_id(0)
    tau = pl.program_id(1)

    @pl.when(tau < GT)
    def _():
        rows = pl.ds(tau * TILE, TILE)
        valid = tv_ref[tau] > 0

        @pl.when(jnp.logical_and(valid, hb == 0))
        def _():
            slot = tau * TILE + jax.lax.broadcasted_iota(
                jnp.int32, (TILE, N), 0)
            p1 = posr_ref[0:1, :]
            p2 = posr_ref[1:2, :]
            P = jnp.logical_or(slot == p1, slot == p2).astype(jnp.bfloat16)
            xs_ref[rows, :] = jnp.dot(P, x_ref[...],
                                      preferred_element_type=jnp.float32
                                      ).astype(jnp.bfloat16)

        @pl.when(valid)
        def _():
            xt = xs_ref[rows, :].astype(jnp.float32)
            h = jnp.dot(xt, up_ref[0], preferred_element_type=jnp.float32)
            h = jnp.square(jnp.maximum(h, 0.0))
            y = jax.lax.dot_general(
                h, dn_ref[0], (((1,), (1,)), ((), ())),
                preferred_element_type=jnp.float32).astype(jnp.bfloat16)

            @pl.when(hb == 0)
            def _():
                acc_ref[rows, :] = y

            @pl.when(hb > 0)
            def _():
                acc_ref[rows, :] += y

        @pl.when(jnp.logical_and(jnp.logical_not(valid), hb == HB - 1))
        def _():
            acc_ref[rows, :] = jnp.zeros((TILE, DIM), jnp.bfloat16)

    @pl.when(jnp.logical_and(tau >= GT, hb == HB - 1))
    def _():
        trows = pl.ds((tau - GT) * TILE, TILE)
        p1 = pc_ref[trows, 0:1]
        p2 = pc_ref[trows, 1:2]
        w1 = tw_ref[trows, 0:1]
        w2 = tw_ref[trows, 1:2]
        slot = jax.lax.broadcasted_iota(jnp.int32, (TILE, NS), 1)
        C = (jnp.where(slot == p1, w1, 0.0)
             + jnp.where(slot == p2, w2, 0.0)).astype(jnp.bfloat16)
        out_ref[...] = sh_ref[...] + jnp.dot(
            C, acc_ref[...], preferred_element_type=jnp.float32)


def _shared_kernel(x_ref, up_ref, dn_ref, out_ref, acc_ref):
    hb = pl.program_id(0)
    t = pl.program_id(1)
    rows = pl.ds(t * TILE, TILE)

    xt = x_ref[rows, :].astype(jnp.float32)
    h = jnp.dot(xt, up_ref[...], preferred_element_type=jnp.float32)
    h = jnp.square(jnp.maximum(h, 0.0))
    y = jax.lax.dot_general(
        h, dn_ref[...], (((1,), (1,)), ((), ())),
        preferred_element_type=jnp.float32)

    @pl.when(hb == 0)
    def _():
        acc_ref[rows, :] = y

    @pl.when(hb > 0)
    def _():
        acc_ref[rows, :] += y

    @pl.when(hb == HB - 1)
    def _():
        out_ref[...] = acc_ref[rows, :]


def kernel(x, router_w, W_shared_up, W_shared_down, W_up, W_down):
    xf = x.reshape(N, DIM)

    counts, p_sum, z_sum, tw, posr, pc, te, tv, x_bf = pl.pallas_call(
        _router_kernel,
        out_shape=[
            jax.ShapeDtypeStruct((1, E), jnp.float32),
            jax.ShapeDtypeStruct((1, E), jnp.float32),
            jax.ShapeDtypeStruct((1, 1), jnp.float32),
            jax.ShapeDtypeStruct((N, TOPK), jnp.float32),
            jax.ShapeDtypeStruct((TOPK, N), jnp.int32),
            jax.ShapeDtypeStruct((N, TOPK), jnp.int32),
            jax.ShapeDtypeStruct((1, 128), jnp.int32),
            jax.ShapeDtypeStruct((1, 128), jnp.int32),
            jax.ShapeDtypeStruct((N, DIM), jnp.bfloat16),
        ],
    )(xf, router_w)

    NT2 = N // TILE
    te_flat = te.reshape(128)[:GT + NT2]
    tv_flat = tv.reshape(128)[:GT + NT2]

    shared_y = pl.pallas_call(
        _shared_kernel,
        grid=(HB, N // TILE),
        in_specs=[
            pl.BlockSpec((N, DIM), lambda hb, t: (0, 0)),
            pl.BlockSpec((DIM, HBLK), lambda hb, t: (0, hb)),
            pl.BlockSpec((DIM, HBLK), lambda hb, t: (0, hb)),
        ],
        out_specs=pl.BlockSpec((TILE, DIM), lambda hb, t: (t, 0)),
        out_shape=jax.ShapeDtypeStruct((N, DIM), jnp.float32),
        scratch_shapes=[pltpu.VMEM((N, DIM), jnp.float32)],
    )(x_bf, W_shared_up, W_shared_down)

    def _tok_map(hb, t, te, tv):
        tt = jnp.where(hb == HB - 1,
                       jnp.clip(t - GT, 0, NT2 - 1), 0)
        return (tt, 0)

    out = pl.pallas_call(
        _group_kernel,
        grid_spec=pltpu.PrefetchScalarGridSpec(
            num_scalar_prefetch=2,
            grid=(HB, GT + NT2),
            in_specs=[
                pl.BlockSpec((TOPK, N), lambda hb, t, te, tv: (0, 0)),
                pl.BlockSpec((N, DIM), lambda hb, t, te, tv: (0, 0)),
                pl.BlockSpec((1, DIM, HBLK),
                             lambda hb, t, te, tv: (te[t], 0, hb)),
                pl.BlockSpec((1, DIM, HBLK),
                             lambda hb, t, te, tv: (te[t], 0, hb)),
                pl.BlockSpec((N, TOPK), lambda hb, t, te, tv: (0, 0)),
                pl.BlockSpec((N, TOPK), lambda hb, t, te, tv: (0, 0)),
                pl.BlockSpec((TILE, DIM), _tok_map),
            ],
            out_specs=pl.BlockSpec((TILE, DIM), _tok_map),
            scratch_shapes=[pltpu.VMEM((NS, DIM), jnp.bfloat16),
                            pltpu.VMEM((NS, DIM), jnp.bfloat16)],
        ),
        out_shape=jax.ShapeDtypeStruct((N, DIM), jnp.float32),
    )(te_flat, tv_flat, posr, x_bf, W_up, W_down, pc, tw, shared_y)

    f = counts[0] / (N * TOPK)
    p_mean = p_sum[0] / N
    lb = E * jnp.sum(f * p_mean)
    z = z_sum[0, 0] / N
    return out.reshape(B, T, DIM), lb, z
